# baseline (speedup 1.0000x reference)
import jax
import jax.numpy as jnp
from jax import lax
from jax.experimental import pallas as pl
from jax.experimental.pallas import tpu as pltpu
from jax.experimental.pallas import tpu_sc as plsc

VOCAB = 100000
EMBED = 128
BATCH = 16384

NC = 2
NS = 16
NW = NC * NS
LANES = 16

ROWS_PER_W = BATCH // 128 // NW
CHUNK = 128
NBUF = 2


def _glove_body(wu_hbm, wv_hbm, in_embed_hbm, in_bias_hbm, out_embed_hbm,
                out_bias_hbm, out_hbm, u_idx, v_idx, u_rows, v_rows,
                u_bias, v_bias, out_buf, psums, sems):
    wid = lax.axis_index("s") * NC + lax.axis_index("c")
    base = wid * ROWS_PER_W

    pltpu.sync_copy(wu_hbm.at[pl.ds(base, ROWS_PER_W)], u_idx)
    pltpu.sync_copy(wv_hbm.at[pl.ds(base, ROWS_PER_W)], v_idx)

    def start(chunk):
        slot = chunk % NBUF
        iu = u_idx.at[chunk]
        iv = v_idx.at[chunk]
        sem = sems.at[slot]
        return (
            pltpu.async_copy(in_embed_hbm.at[iu], u_rows.at[slot], sem),
            pltpu.async_copy(out_embed_hbm.at[iv], v_rows.at[slot], sem),
        )

    row_ids = lax.iota(jnp.int32, LANES)
    pending = start(0)

    bias_dmas = []
    for c in range(ROWS_PER_W):
        bias_dmas.append(pltpu.async_copy(
            in_bias_hbm.at[u_idx.at[c]], u_bias.at[c], sems.at[NBUF]))
        bias_dmas.append(pltpu.async_copy(
            out_bias_hbm.at[v_idx.at[c]], v_bias.at[c], sems.at[NBUF]))
    for d in bias_dmas:
        d.wait()

    for chunk in range(ROWS_PER_W):
        for d in pending:
            d.wait()
        if chunk + 1 < ROWS_PER_W:
            pending = start(chunk + 1)
        slot = chunk % NBUF
        ur = u_rows.at[slot]
        vr = v_rows.at[slot]

        def group_body(g, carry, *, chunk=chunk, ur=ur, vr=vr):
            for h in range(2):
                for p in range(LANES):
                    pair = (2 * g + h) * LANES + p
                    ms = [ur[pair, pl.ds(k * LANES, LANES)]
                          * vr[pair, pl.ds(k * LANES, LANES)]
                          for k in range(EMBED // LANES)]
                    while len(ms) > 1:
                        ms = [ms[i] + ms[i + 1] for i in range(0, len(ms), 2)]
                    psums[h * LANES + p, pl.ds(0, LANES)] = plsc.cumsum(ms[0])
            col15 = jnp.full((LANES,), LANES - 1, jnp.int32)
            for h in range(2):
                tot = plsc.load_gather(
                    psums, [h * LANES + row_ids, col15])
                sl = pl.ds((2 * g + h) * LANES, LANES)
                out_buf[chunk, sl] = (tot + u_bias[chunk, sl]
                                      + v_bias[chunk, sl])
            return carry

        lax.fori_loop(0, CHUNK // (2 * LANES), group_body, 0)

    pltpu.sync_copy(out_buf, out_hbm.at[pl.ds(base, ROWS_PER_W)])


@jax.jit
def _glove_sc(wu, wv, in_embed, in_bias, out_embed, out_bias):
    kern = pl.kernel(
        _glove_body,
        out_type=jax.ShapeDtypeStruct((128, 128), jnp.float32),
        mesh=plsc.VectorSubcoreMesh(core_axis_name="c", subcore_axis_name="s"),
        compiler_params=pltpu.CompilerParams(needs_layout_passes=False),
        scratch_types=[
            pltpu.VMEM((ROWS_PER_W, 128), jnp.int32),
            pltpu.VMEM((ROWS_PER_W, 128), jnp.int32),
            pltpu.VMEM((NBUF, CHUNK, EMBED), jnp.float32),
            pltpu.VMEM((NBUF, CHUNK, EMBED), jnp.float32),
            pltpu.VMEM((ROWS_PER_W, CHUNK), jnp.float32),
            pltpu.VMEM((ROWS_PER_W, CHUNK), jnp.float32),
            pltpu.VMEM((ROWS_PER_W, 128), jnp.float32),
            pltpu.VMEM((2 * LANES, LANES), jnp.float32),
            pltpu.SemaphoreType.DMA((NBUF + 1,)),
        ],
    )
    return kern(wu, wv, in_embed, in_bias, out_embed, out_bias)


def kernel(word_u, word_v, in_embed, in_bias, out_embed, out_bias):
    wu = word_u.reshape(128, 128)
    wv = word_v.reshape(128, 128)
    ib = in_bias.reshape(VOCAB)
    ob = out_bias.reshape(VOCAB)
    out = _glove_sc(wu, wv, in_embed, ib, out_embed, ob)
    return out.reshape(BATCH)


# speedup vs baseline: 1.0949x; 1.0949x over previous
import jax
import jax.numpy as jnp
from jax import lax
from jax.experimental import pallas as pl
from jax.experimental.pallas import tpu as pltpu
from jax.experimental.pallas import tpu_sc as plsc

VOCAB = 100000
EMBED = 128
BATCH = 16384

NC = 2
NS = 16
NW = NC * NS
LANES = 16

ROWS_PER_W = BATCH // 128 // NW
CHUNK = 128
NBUF = 2


def _glove_body(wu_hbm, wv_hbm, in_embed_hbm, in_bias_hbm, out_embed_hbm,
                out_bias_hbm, out_hbm, u_idx, v_idx, u_rows, v_rows,
                u_bias, v_bias, out_buf, psums, sems):
    wid = lax.axis_index("s") * NC + lax.axis_index("c")
    base = wid * ROWS_PER_W

    pltpu.sync_copy(wu_hbm.at[pl.ds(base, ROWS_PER_W)], u_idx)
    pltpu.sync_copy(wv_hbm.at[pl.ds(base, ROWS_PER_W)], v_idx)

    def start(chunk):
        slot = chunk % NBUF
        iu = u_idx.at[chunk]
        iv = v_idx.at[chunk]
        sem = sems.at[slot]
        return (
            pltpu.async_copy(in_embed_hbm.at[iu], u_rows.at[slot], sem),
            pltpu.async_copy(out_embed_hbm.at[iv], v_rows.at[slot], sem),
        )

    row_ids = lax.iota(jnp.int32, LANES)
    pending = start(0)

    bias_dmas = []
    for c in range(ROWS_PER_W):
        bias_dmas.append(pltpu.async_copy(
            in_bias_hbm.at[u_idx.at[c]], u_bias.at[c], sems.at[NBUF]))
        bias_dmas.append(pltpu.async_copy(
            out_bias_hbm.at[v_idx.at[c]], v_bias.at[c], sems.at[NBUF]))
    for d in bias_dmas:
        d.wait()

    for chunk in range(ROWS_PER_W):
        for d in pending:
            d.wait()
        if chunk + 1 < ROWS_PER_W:
            pending = start(chunk + 1)
        slot = chunk % NBUF
        ur = u_rows.at[slot]
        vr = v_rows.at[slot]

        def group_body(g, carry, *, chunk=chunk, ur=ur, vr=vr):
            for p in range(LANES):
                pair = g * LANES + p
                ms = [ur[pair, pl.ds(k * LANES, LANES)]
                      * vr[pair, pl.ds(k * LANES, LANES)]
                      for k in range(EMBED // LANES)]
                while len(ms) > 1:
                    ms = [ms[i] + ms[i + 1] for i in range(0, len(ms), 2)]
                psums[p, pl.ds(0, LANES)] = ms[0]
            cols = [plsc.load_gather(
                psums, [row_ids, jnp.full((LANES,), j, jnp.int32)])
                for j in range(LANES)]
            while len(cols) > 1:
                cols = [cols[i] + cols[i + 1] for i in range(0, len(cols), 2)]
            sl = pl.ds(g * LANES, LANES)
            out_buf[chunk, sl] = (cols[0] + u_bias[chunk, sl]
                                  + v_bias[chunk, sl])
            return carry

        lax.fori_loop(0, CHUNK // LANES, group_body, 0)

    pltpu.sync_copy(out_buf, out_hbm.at[pl.ds(base, ROWS_PER_W)])


@jax.jit
def _glove_sc(wu, wv, in_embed, in_bias, out_embed, out_bias):
    kern = pl.kernel(
        _glove_body,
        out_type=jax.ShapeDtypeStruct((128, 128), jnp.float32),
        mesh=plsc.VectorSubcoreMesh(core_axis_name="c", subcore_axis_name="s"),
        compiler_params=pltpu.CompilerParams(needs_layout_passes=False),
        scratch_types=[
            pltpu.VMEM((ROWS_PER_W, 128), jnp.int32),
            pltpu.VMEM((ROWS_PER_W, 128), jnp.int32),
            pltpu.VMEM((NBUF, CHUNK, EMBED), jnp.float32),
            pltpu.VMEM((NBUF, CHUNK, EMBED), jnp.float32),
            pltpu.VMEM((ROWS_PER_W, CHUNK), jnp.float32),
            pltpu.VMEM((ROWS_PER_W, CHUNK), jnp.float32),
            pltpu.VMEM((ROWS_PER_W, 128), jnp.float32),
            pltpu.VMEM((LANES, LANES + 1), jnp.float32),
            pltpu.SemaphoreType.DMA((NBUF + 1,)),
        ],
    )
    return kern(wu, wv, in_embed, in_bias, out_embed, out_bias)


def kernel(word_u, word_v, in_embed, in_bias, out_embed, out_bias):
    wu = word_u.reshape(128, 128)
    wv = word_v.reshape(128, 128)
    ib = in_bias.reshape(VOCAB)
    ob = out_bias.reshape(VOCAB)
    out = _glove_sc(wu, wv, in_embed, ib, out_embed, ob)
    return out.reshape(BATCH)


# speedup vs baseline: 1.1104x; 1.0141x over previous
import jax
import jax.numpy as jnp
from jax import lax
from jax.experimental import pallas as pl
from jax.experimental.pallas import tpu as pltpu
from jax.experimental.pallas import tpu_sc as plsc

VOCAB = 100000
EMBED = 128
BATCH = 16384

NC = 2
NS = 16
NW = NC * NS
LANES = 16

ROWS_PER_W = BATCH // 128 // NW
CHUNK = 128
NBUF = 2


def _glove_body(wu_hbm, wv_hbm, in_embed_hbm, in_bias_hbm, out_embed_hbm,
                out_bias_hbm, out_hbm, u_idx, v_idx, u_rows, v_rows,
                u_bias, v_bias, out_buf, psums, sems):
    wid = lax.axis_index("s") * NC + lax.axis_index("c")
    base = wid * ROWS_PER_W

    pltpu.sync_copy(wu_hbm.at[pl.ds(base, ROWS_PER_W)], u_idx)
    pltpu.sync_copy(wv_hbm.at[pl.ds(base, ROWS_PER_W)], v_idx)

    def start(chunk):
        slot = chunk % NBUF
        iu = u_idx.at[chunk]
        iv = v_idx.at[chunk]
        sem = sems.at[slot]
        return (
            pltpu.async_copy(in_embed_hbm.at[iu], u_rows.at[slot], sem),
            pltpu.async_copy(out_embed_hbm.at[iv], v_rows.at[slot], sem),
            pltpu.async_copy(in_bias_hbm.at[iu], u_bias.at[chunk], sem),
            pltpu.async_copy(out_bias_hbm.at[iv], v_bias.at[chunk], sem),
        )

    row_ids = lax.iota(jnp.int32, LANES)
    pending = start(0)

    for chunk in range(ROWS_PER_W):
        for d in pending:
            d.wait()
        if chunk + 1 < ROWS_PER_W:
            pending = start(chunk + 1)
        slot = chunk % NBUF
        ur = u_rows.at[slot]
        vr = v_rows.at[slot]

        def group_body(g, carry, *, chunk=chunk, ur=ur, vr=vr):
            for p in range(LANES):
                pair = g * LANES + p
                ms = [ur[pair, pl.ds(k * LANES, LANES)]
                      * vr[pair, pl.ds(k * LANES, LANES)]
                      for k in range(EMBED // LANES)]
                while len(ms) > 1:
                    ms = [ms[i] + ms[i + 1] for i in range(0, len(ms), 2)]
                psums[p, pl.ds(0, LANES)] = ms[0]
            cols = [plsc.load_gather(
                psums, [row_ids, jnp.full((LANES,), j, jnp.int32)])
                for j in range(LANES)]
            while len(cols) > 1:
                cols = [cols[i] + cols[i + 1] for i in range(0, len(cols), 2)]
            sl = pl.ds(g * LANES, LANES)
            out_buf[chunk, sl] = (cols[0] + u_bias[chunk, sl]
                                  + v_bias[chunk, sl])
            return carry

        lax.fori_loop(0, CHUNK // LANES, group_body, 0)

    pltpu.sync_copy(out_buf, out_hbm.at[pl.ds(base, ROWS_PER_W)])


@jax.jit
def _glove_sc(wu, wv, in_embed, in_bias, out_embed, out_bias):
    kern = pl.kernel(
        _glove_body,
        out_type=jax.ShapeDtypeStruct((128, 128), jnp.float32),
        mesh=plsc.VectorSubcoreMesh(core_axis_name="c", subcore_axis_name="s"),
        compiler_params=pltpu.CompilerParams(needs_layout_passes=False),
        scratch_types=[
            pltpu.VMEM((ROWS_PER_W, 128), jnp.int32),
            pltpu.VMEM((ROWS_PER_W, 128), jnp.int32),
            pltpu.VMEM((NBUF, CHUNK, EMBED), jnp.float32),
            pltpu.VMEM((NBUF, CHUNK, EMBED), jnp.float32),
            pltpu.VMEM((ROWS_PER_W, CHUNK), jnp.float32),
            pltpu.VMEM((ROWS_PER_W, CHUNK), jnp.float32),
            pltpu.VMEM((ROWS_PER_W, 128), jnp.float32),
            pltpu.VMEM((LANES, LANES + 1), jnp.float32),
            pltpu.SemaphoreType.DMA((NBUF + 1,)),
        ],
    )
    return kern(wu, wv, in_embed, in_bias, out_embed, out_bias)


def kernel(word_u, word_v, in_embed, in_bias, out_embed, out_bias):
    wu = word_u.reshape(128, 128)
    wv = word_v.reshape(128, 128)
    ib = in_bias.reshape(VOCAB)
    ob = out_bias.reshape(VOCAB)
    out = _glove_sc(wu, wv, in_embed, ib, out_embed, ob)
    return out.reshape(BATCH)
